# baseline (speedup 1.0000x reference)
import functools
import math

import jax
import jax.numpy as jnp
import numpy as np
from jax.experimental import pallas as pl
from jax.experimental.pallas import tpu as pltpu

_ROT_A = (13, 15, 26, 6)
_ROT_B = (17, 29, 16, 24)
_TINY = np.float32(np.finfo(np.float32).tiny)


def _np_threefry2x32(k0, k1, x0, x1):
    x0 = np.asarray(x0, np.uint32).copy()
    x1 = np.asarray(x1, np.uint32).copy()
    ks = [np.uint32(k0), np.uint32(k1),
          np.uint32(np.uint32(k0) ^ np.uint32(k1) ^ np.uint32(0x1BD11BDA))]
    rots = [_ROT_A, _ROT_B]
    x0 = (x0 + ks[0]).astype(np.uint32)
    x1 = (x1 + ks[1]).astype(np.uint32)
    for i in range(5):
        for r in rots[i % 2]:
            x0 = (x0 + x1).astype(np.uint32)
            x1 = ((x1 << np.uint32(r)) | (x1 >> np.uint32(32 - r))).astype(np.uint32)
            x1 = (x1 ^ x0).astype(np.uint32)
        x0 = (x0 + ks[(i + 1) % 3]).astype(np.uint32)
        x1 = (x1 + ks[(i + 2) % 3] + np.uint32(i + 1)).astype(np.uint32)
    return x0, x1


def _field_keys(n_fields):
    idx = np.arange(n_fields, dtype=np.uint32)
    o0, o1 = _np_threefry2x32(0, 42, np.zeros(n_fields, np.uint32), idx)
    return o0, o1


def _tf_rounds(x0, x1, k0, k1):
    k2 = k0 ^ k1 ^ jnp.uint32(0x1BD11BDA)
    ks = (k0, k1, k2)
    x0 = x0 + ks[0]
    x1 = x1 + ks[1]
    for i in range(5):
        rots = _ROT_A if i % 2 == 0 else _ROT_B
        for r in rots:
            x0 = x0 + x1
            x1 = (x1 << jnp.uint32(r)) | (x1 >> jnp.uint32(32 - r))
            x1 = x1 ^ x0
        x0 = x0 + ks[(i + 1) % 3]
        x1 = x1 + ks[(i + 2) % 3] + jnp.uint32(i + 1)
    return x0, x1


def _sample_kernel(logits_ref, k0_ref, k1_ref, act_ref, lp_ref,
                   accv_s, acci_s, accl_s, m_s, s_s,
                   *, block, chunk, vocab, nb):
    b = pl.program_id(1)
    rows = logits_ref.shape[0]
    nchunks = block // chunk
    base = b * block
    neg_inf = jnp.float32(-jnp.inf)
    lane0 = jax.lax.broadcasted_iota(jnp.int32, (rows, chunk), 1)
    k0 = k0_ref[...]
    k1 = k1_ref[...]

    @pl.when(b == 0)
    def _init():
        accv_s[...] = jnp.full((rows, chunk), neg_inf, jnp.float32)
        acci_s[...] = jnp.zeros((rows, chunk), jnp.int32)
        accl_s[...] = jnp.zeros((rows, chunk), jnp.float32)
        m_s[...] = jnp.full((rows, 1), neg_inf, jnp.float32)
        s_s[...] = jnp.zeros((rows, 1), jnp.float32)

    def body(c, carry):
        accv, acci, accl, m, s = carry
        start = c * chunk
        l = logits_ref[:, pl.ds(start, chunk)]
        gidx = lane0 + (base + start)
        valid = gidx < vocab

        o0, o1 = _tf_rounds(jnp.zeros((rows, chunk), jnp.uint32),
                            gidx.astype(jnp.uint32), k0, k1)
        bits = o0 ^ o1
        uf = pltpu.bitcast((bits >> jnp.uint32(9)) | jnp.uint32(0x3F800000),
                           jnp.float32) - jnp.float32(1.0)
        u = jnp.maximum(_TINY, uf + _TINY)
        g = -jnp.log(-jnp.log(u))

        lm = jnp.where(valid, l, neg_inf)
        v = jnp.where(valid, l + g, neg_inf)

        take = v > accv
        accv = jnp.maximum(v, accv)
        acci = jnp.where(take, gidx, acci)
        accl = jnp.where(take, l, accl)

        blm = jnp.max(lm, axis=1, keepdims=True)
        bs = jnp.sum(jnp.where(valid, jnp.exp(lm - blm), 0.0), axis=1,
                     keepdims=True)
        mn = jnp.maximum(m, blm)
        s = s * jnp.exp(m - mn) + bs * jnp.exp(blm - mn)
        return accv, acci, accl, mn, s

    carry0 = (accv_s[...], acci_s[...], accl_s[...], m_s[...], s_s[...])
    accv, acci, accl, m, s = jax.lax.fori_loop(0, nchunks, body, carry0)
    accv_s[...] = accv
    acci_s[...] = acci
    accl_s[...] = accl
    m_s[...] = m
    s_s[...] = s

    @pl.when(b == nb - 1)
    def _emit():
        bv = jnp.max(accv, axis=1, keepdims=True)
        tied = accv == bv
        big = jnp.int32(2**31 - 1)
        bi = jnp.min(jnp.where(tied, acci, big), axis=1, keepdims=True)
        win = tied & (acci == bi)
        bl = jnp.max(jnp.where(win, accl, neg_inf), axis=1, keepdims=True)
        act_ref[...] = bi
        lp_ref[...] = bl - (m + jnp.log(s))


def kernel(logits):
    n_fields, vocab = logits.shape
    rows = 8
    block = 8192
    chunk = 512
    nfb = math.ceil(n_fields / rows)
    nb = math.ceil(vocab / block)
    nfp = nfb * rows

    k0np, k1np = _field_keys(n_fields)
    k0 = jnp.asarray(np.pad(k0np, (0, nfp - n_fields)).reshape(nfp, 1))
    k1 = jnp.asarray(np.pad(k1np, (0, nfp - n_fields)).reshape(nfp, 1))

    act, lp = pl.pallas_call(
        functools.partial(_sample_kernel, block=block, chunk=chunk,
                          vocab=vocab, nb=nb),
        grid=(nfb, nb),
        in_specs=[
            pl.BlockSpec((rows, block), lambda f, b: (f, b)),
            pl.BlockSpec((rows, 1), lambda f, b: (f, 0)),
            pl.BlockSpec((rows, 1), lambda f, b: (f, 0)),
        ],
        out_specs=[
            pl.BlockSpec((rows, 1), lambda f, b: (f, 0)),
            pl.BlockSpec((rows, 1), lambda f, b: (f, 0)),
        ],
        out_shape=[
            jax.ShapeDtypeStruct((nfp, 1), jnp.int32),
            jax.ShapeDtypeStruct((nfp, 1), jnp.float32),
        ],
        scratch_shapes=[
            pltpu.VMEM((rows, chunk), jnp.float32),
            pltpu.VMEM((rows, chunk), jnp.int32),
            pltpu.VMEM((rows, chunk), jnp.float32),
            pltpu.VMEM((rows, 1), jnp.float32),
            pltpu.VMEM((rows, 1), jnp.float32),
        ],
        compiler_params=pltpu.CompilerParams(
            dimension_semantics=("parallel", "arbitrary")),
    )(logits, k0, k1)

    action = act[:n_fields, 0]
    log_prob = lp[:n_fields, 0].sum()
    return (action, log_prob, jnp.float32(1.0))


# speedup vs baseline: 1.9291x; 1.9291x over previous
import functools
import math

import jax
import jax.numpy as jnp
import numpy as np
from jax.experimental import pallas as pl
from jax.experimental.pallas import tpu as pltpu

_ROT_A = (13, 15, 26, 6)
_ROT_B = (17, 29, 16, 24)
_TINY = np.float32(np.finfo(np.float32).tiny)


def _np_threefry2x32(k0, k1, x0, x1):
    x0 = np.asarray(x0, np.uint32).copy()
    x1 = np.asarray(x1, np.uint32).copy()
    ks = [np.uint32(k0), np.uint32(k1),
          np.uint32(np.uint32(k0) ^ np.uint32(k1) ^ np.uint32(0x1BD11BDA))]
    rots = [_ROT_A, _ROT_B]
    x0 = (x0 + ks[0]).astype(np.uint32)
    x1 = (x1 + ks[1]).astype(np.uint32)
    for i in range(5):
        for r in rots[i % 2]:
            x0 = (x0 + x1).astype(np.uint32)
            x1 = ((x1 << np.uint32(r)) | (x1 >> np.uint32(32 - r))).astype(np.uint32)
            x1 = (x1 ^ x0).astype(np.uint32)
        x0 = (x0 + ks[(i + 1) % 3]).astype(np.uint32)
        x1 = (x1 + ks[(i + 2) % 3] + np.uint32(i + 1)).astype(np.uint32)
    return x0, x1


def _field_keys(n_fields):
    idx = np.arange(n_fields, dtype=np.uint32)
    o0, o1 = _np_threefry2x32(0, 42, np.zeros(n_fields, np.uint32), idx)
    return o0, o1


def _tf_rounds(x0, x1, k0, k1):
    k2 = k0 ^ k1 ^ jnp.uint32(0x1BD11BDA)
    ks = (k0, k1, k2)
    x0 = x0 + ks[0]
    x1 = x1 + ks[1]
    for i in range(5):
        rots = _ROT_A if i % 2 == 0 else _ROT_B
        for r in rots:
            x0 = x0 + x1
            x1 = (x1 << jnp.uint32(r)) | (x1 >> jnp.uint32(32 - r))
            x1 = x1 ^ x0
        x0 = x0 + ks[(i + 1) % 3]
        x1 = x1 + ks[(i + 2) % 3] + jnp.uint32(i + 1)
    return x0, x1


def _sample_kernel(logits_ref, k0_ref, k1_ref, act_ref, lp_ref,
                   accv_s, acci_s, accl_s, m_s, s_s,
                   *, block, chunk, vocab, nb):
    b = pl.program_id(1)
    rows = logits_ref.shape[0]
    nchunks = block // chunk
    base = b * block
    neg_inf = jnp.float32(-jnp.inf)
    lane0 = jax.lax.broadcasted_iota(jnp.int32, (rows, chunk), 1)
    k0 = k0_ref[...]
    k1 = k1_ref[...]

    @pl.when(b == 0)
    def _init():
        accv_s[...] = jnp.full((rows, chunk), neg_inf, jnp.float32)
        acci_s[...] = jnp.zeros((rows, chunk), jnp.int32)
        accl_s[...] = jnp.zeros((rows, chunk), jnp.float32)
        m_s[...] = jnp.full((rows, 1), neg_inf, jnp.float32)
        s_s[...] = jnp.zeros((rows, 1), jnp.float32)

    def body(c, carry):
        accv, acci, accl, m, s = carry
        start = c * chunk
        l = logits_ref[:, pl.ds(start, chunk)]
        gidx = lane0 + (base + start)
        valid = gidx < vocab

        o0, o1 = _tf_rounds(jnp.zeros((rows, chunk), jnp.uint32),
                            gidx.astype(jnp.uint32), k0, k1)
        bits = o0 ^ o1
        uf = pltpu.bitcast((bits >> jnp.uint32(9)) | jnp.uint32(0x3F800000),
                           jnp.float32) - jnp.float32(1.0)
        u = jnp.maximum(_TINY, uf + _TINY)
        g = -jnp.log(-jnp.log(u))

        lm = jnp.where(valid, l, neg_inf)
        v = jnp.where(valid, l + g, neg_inf)

        take = v > accv
        accv = jnp.maximum(v, accv)
        acci = jnp.where(take, gidx, acci)
        accl = jnp.where(take, l, accl)

        blm = jnp.max(lm, axis=1, keepdims=True)
        bs = jnp.sum(jnp.where(valid, jnp.exp(lm - blm), 0.0), axis=1,
                     keepdims=True)
        mn = jnp.maximum(m, blm)
        s = s * jnp.exp(m - mn) + bs * jnp.exp(blm - mn)
        return accv, acci, accl, mn, s

    carry0 = (accv_s[...], acci_s[...], accl_s[...], m_s[...], s_s[...])
    accv, acci, accl, m, s = jax.lax.fori_loop(0, nchunks, body, carry0,
                                               unroll=4)
    accv_s[...] = accv
    acci_s[...] = acci
    accl_s[...] = accl
    m_s[...] = m
    s_s[...] = s

    @pl.when(b == nb - 1)
    def _emit():
        bv = jnp.max(accv, axis=1, keepdims=True)
        tied = accv == bv
        big = jnp.int32(2**31 - 1)
        bi = jnp.min(jnp.where(tied, acci, big), axis=1, keepdims=True)
        win = tied & (acci == bi)
        bl = jnp.max(jnp.where(win, accl, neg_inf), axis=1, keepdims=True)
        act_ref[...] = bi
        lp_ref[...] = bl - (m + jnp.log(s))


def kernel(logits):
    n_fields, vocab = logits.shape
    rows = 8
    block = 8192
    chunk = 512
    nfb = math.ceil(n_fields / rows)
    nb = math.ceil(vocab / block)
    nfp = nfb * rows

    k0np, k1np = _field_keys(n_fields)
    k0 = jnp.asarray(np.pad(k0np, (0, nfp - n_fields)).reshape(nfp, 1))
    k1 = jnp.asarray(np.pad(k1np, (0, nfp - n_fields)).reshape(nfp, 1))

    act, lp = pl.pallas_call(
        functools.partial(_sample_kernel, block=block, chunk=chunk,
                          vocab=vocab, nb=nb),
        grid=(nfb, nb),
        in_specs=[
            pl.BlockSpec((rows, block), lambda f, b: (f, b)),
            pl.BlockSpec((rows, 1), lambda f, b: (f, 0)),
            pl.BlockSpec((rows, 1), lambda f, b: (f, 0)),
        ],
        out_specs=[
            pl.BlockSpec((rows, 1), lambda f, b: (f, 0)),
            pl.BlockSpec((rows, 1), lambda f, b: (f, 0)),
        ],
        out_shape=[
            jax.ShapeDtypeStruct((nfp, 1), jnp.int32),
            jax.ShapeDtypeStruct((nfp, 1), jnp.float32),
        ],
        scratch_shapes=[
            pltpu.VMEM((rows, chunk), jnp.float32),
            pltpu.VMEM((rows, chunk), jnp.int32),
            pltpu.VMEM((rows, chunk), jnp.float32),
            pltpu.VMEM((rows, 1), jnp.float32),
            pltpu.VMEM((rows, 1), jnp.float32),
        ],
        compiler_params=pltpu.CompilerParams(
            dimension_semantics=("parallel", "arbitrary")),
    )(logits, k0, k1)

    action = act[:n_fields, 0]
    log_prob = lp[:n_fields, 0].sum()
    return (action, log_prob, jnp.float32(1.0))


# speedup vs baseline: 2.1456x; 1.1122x over previous
import functools
import math

import jax
import jax.numpy as jnp
import numpy as np
from jax.experimental import pallas as pl
from jax.experimental.pallas import tpu as pltpu

_ROT_A = (13, 15, 26, 6)
_ROT_B = (17, 29, 16, 24)
_TINY = np.float32(np.finfo(np.float32).tiny)


def _np_threefry2x32(k0, k1, x0, x1):
    x0 = np.asarray(x0, np.uint32).copy()
    x1 = np.asarray(x1, np.uint32).copy()
    ks = [np.uint32(k0), np.uint32(k1),
          np.uint32(np.uint32(k0) ^ np.uint32(k1) ^ np.uint32(0x1BD11BDA))]
    rots = [_ROT_A, _ROT_B]
    x0 = (x0 + ks[0]).astype(np.uint32)
    x1 = (x1 + ks[1]).astype(np.uint32)
    for i in range(5):
        for r in rots[i % 2]:
            x0 = (x0 + x1).astype(np.uint32)
            x1 = ((x1 << np.uint32(r)) | (x1 >> np.uint32(32 - r))).astype(np.uint32)
            x1 = (x1 ^ x0).astype(np.uint32)
        x0 = (x0 + ks[(i + 1) % 3]).astype(np.uint32)
        x1 = (x1 + ks[(i + 2) % 3] + np.uint32(i + 1)).astype(np.uint32)
    return x0, x1


def _field_keys(n_fields):
    idx = np.arange(n_fields, dtype=np.uint32)
    o0, o1 = _np_threefry2x32(0, 42, np.zeros(n_fields, np.uint32), idx)
    return o0, o1


def _tf_rounds(x0, x1, k0, k1):
    k2 = k0 ^ k1 ^ jnp.uint32(0x1BD11BDA)
    ks = (k0, k1, k2)
    x0 = x0 + ks[0]
    x1 = x1 + ks[1]
    for i in range(5):
        rots = _ROT_A if i % 2 == 0 else _ROT_B
        for r in rots:
            x0 = x0 + x1
            x1 = (x1 << jnp.uint32(r)) | (x1 >> jnp.uint32(32 - r))
            x1 = x1 ^ x0
        x0 = x0 + ks[(i + 1) % 3]
        x1 = x1 + ks[(i + 2) % 3] + jnp.uint32(i + 1)
    return x0, x1


def _sample_kernel(logits_ref, k0_ref, k1_ref, act_ref, lp_ref,
                   accv_s, acci_s, accl_s, m_s, s_s,
                   *, block, chunk, vocab, nb):
    b = pl.program_id(1)
    rows = logits_ref.shape[0]
    nchunks = block // chunk
    base = b * block
    neg_inf = jnp.float32(-jnp.inf)
    lane0 = jax.lax.broadcasted_iota(jnp.int32, (rows, chunk), 1)
    k0 = k0_ref[...]
    k1 = k1_ref[...]

    @pl.when(b == 0)
    def _init():
        accv_s[...] = jnp.full((rows, chunk), neg_inf, jnp.float32)
        acci_s[...] = jnp.zeros((rows, chunk), jnp.int32)
        accl_s[...] = jnp.zeros((rows, chunk), jnp.float32)
        m_s[...] = jnp.full((rows, 1), neg_inf, jnp.float32)
        s_s[...] = jnp.zeros((rows, 1), jnp.float32)

    def body(c, carry):
        accv, acci, accl, m, s = carry
        start = c * chunk
        l = logits_ref[:, pl.ds(start, chunk)]
        gidx = lane0 + (base + start)
        valid = gidx < vocab

        o0, o1 = _tf_rounds(jnp.zeros((rows, chunk), jnp.uint32),
                            gidx.astype(jnp.uint32), k0, k1)
        bits = o0 ^ o1
        uf = pltpu.bitcast((bits >> jnp.uint32(9)) | jnp.uint32(0x3F800000),
                           jnp.float32) - jnp.float32(1.0)
        u = jnp.maximum(_TINY, uf + _TINY)
        g = -jnp.log(-jnp.log(u))

        lm = jnp.where(valid, l, neg_inf)
        v = jnp.where(valid, l + g, neg_inf)

        take = v > accv
        accv = jnp.maximum(v, accv)
        acci = jnp.where(take, gidx, acci)
        accl = jnp.where(take, l, accl)

        blm = jnp.max(lm, axis=1, keepdims=True)
        bs = jnp.sum(jnp.where(valid, jnp.exp(lm - blm), 0.0), axis=1,
                     keepdims=True)
        mn = jnp.maximum(m, blm)
        s = s * jnp.exp(m - mn) + bs * jnp.exp(blm - mn)
        return accv, acci, accl, mn, s

    carry0 = (accv_s[...], acci_s[...], accl_s[...], m_s[...], s_s[...])
    accv, acci, accl, m, s = jax.lax.fori_loop(0, nchunks, body, carry0,
                                               unroll=8)
    accv_s[...] = accv
    acci_s[...] = acci
    accl_s[...] = accl
    m_s[...] = m
    s_s[...] = s

    @pl.when(b == nb - 1)
    def _emit():
        bv = jnp.max(accv, axis=1, keepdims=True)
        tied = accv == bv
        big = jnp.int32(2**31 - 1)
        bi = jnp.min(jnp.where(tied, acci, big), axis=1, keepdims=True)
        win = tied & (acci == bi)
        bl = jnp.max(jnp.where(win, accl, neg_inf), axis=1, keepdims=True)
        act_ref[...] = bi
        lp_ref[...] = bl - (m + jnp.log(s))


def kernel(logits):
    n_fields, vocab = logits.shape
    rows = 8
    block = 8192
    chunk = 512
    nfb = math.ceil(n_fields / rows)
    nb = math.ceil(vocab / block)
    nfp = nfb * rows

    k0np, k1np = _field_keys(n_fields)
    k0 = jnp.asarray(np.pad(k0np, (0, nfp - n_fields)).reshape(nfp, 1))
    k1 = jnp.asarray(np.pad(k1np, (0, nfp - n_fields)).reshape(nfp, 1))

    act, lp = pl.pallas_call(
        functools.partial(_sample_kernel, block=block, chunk=chunk,
                          vocab=vocab, nb=nb),
        grid=(nfb, nb),
        in_specs=[
            pl.BlockSpec((rows, block), lambda f, b: (f, b)),
            pl.BlockSpec((rows, 1), lambda f, b: (f, 0)),
            pl.BlockSpec((rows, 1), lambda f, b: (f, 0)),
        ],
        out_specs=[
            pl.BlockSpec((rows, 1), lambda f, b: (f, 0)),
            pl.BlockSpec((rows, 1), lambda f, b: (f, 0)),
        ],
        out_shape=[
            jax.ShapeDtypeStruct((nfp, 1), jnp.int32),
            jax.ShapeDtypeStruct((nfp, 1), jnp.float32),
        ],
        scratch_shapes=[
            pltpu.VMEM((rows, chunk), jnp.float32),
            pltpu.VMEM((rows, chunk), jnp.int32),
            pltpu.VMEM((rows, chunk), jnp.float32),
            pltpu.VMEM((rows, 1), jnp.float32),
            pltpu.VMEM((rows, 1), jnp.float32),
        ],
        compiler_params=pltpu.CompilerParams(
            dimension_semantics=("parallel", "arbitrary")),
    )(logits, k0, k1)

    action = act[:n_fields, 0]
    log_prob = lp[:n_fields, 0].sum()
    return (action, log_prob, jnp.float32(1.0))


# speedup vs baseline: 2.2976x; 1.0709x over previous
import functools
import math

import jax
import jax.numpy as jnp
import numpy as np
from jax.experimental import pallas as pl
from jax.experimental.pallas import tpu as pltpu

_ROT_A = (13, 15, 26, 6)
_ROT_B = (17, 29, 16, 24)
_TINY = np.float32(np.finfo(np.float32).tiny)


def _np_threefry2x32(k0, k1, x0, x1):
    x0 = np.asarray(x0, np.uint32).copy()
    x1 = np.asarray(x1, np.uint32).copy()
    ks = [np.uint32(k0), np.uint32(k1),
          np.uint32(np.uint32(k0) ^ np.uint32(k1) ^ np.uint32(0x1BD11BDA))]
    rots = [_ROT_A, _ROT_B]
    x0 = (x0 + ks[0]).astype(np.uint32)
    x1 = (x1 + ks[1]).astype(np.uint32)
    for i in range(5):
        for r in rots[i % 2]:
            x0 = (x0 + x1).astype(np.uint32)
            x1 = ((x1 << np.uint32(r)) | (x1 >> np.uint32(32 - r))).astype(np.uint32)
            x1 = (x1 ^ x0).astype(np.uint32)
        x0 = (x0 + ks[(i + 1) % 3]).astype(np.uint32)
        x1 = (x1 + ks[(i + 2) % 3] + np.uint32(i + 1)).astype(np.uint32)
    return x0, x1


def _field_keys(n_fields):
    idx = np.arange(n_fields, dtype=np.uint32)
    o0, o1 = _np_threefry2x32(0, 42, np.zeros(n_fields, np.uint32), idx)
    return o0, o1


def _tf_rounds(x0, x1, k0, k1):
    k2 = k0 ^ k1 ^ jnp.uint32(0x1BD11BDA)
    ks = (k0, k1, k2)
    x0 = x0 + ks[0]
    x1 = x1 + ks[1]
    for i in range(5):
        rots = _ROT_A if i % 2 == 0 else _ROT_B
        for r in rots:
            x0 = x0 + x1
            x1 = (x1 << jnp.uint32(r)) | (x1 >> jnp.uint32(32 - r))
            x1 = x1 ^ x0
        x0 = x0 + ks[(i + 1) % 3]
        x1 = x1 + ks[(i + 2) % 3] + jnp.uint32(i + 1)
    return x0, x1


def _sample_kernel(logits_ref, k0_ref, k1_ref, act_ref, lp_ref,
                   accv_s, acci_s, accl_s, m_s, s_s,
                   *, block, chunk, vocab, nb):
    b = pl.program_id(1)
    rows = logits_ref.shape[0]
    nchunks = block // chunk
    base = b * block
    neg_inf = jnp.float32(-jnp.inf)
    lane0 = jax.lax.broadcasted_iota(jnp.int32, (rows, chunk), 1)
    k0 = k0_ref[...]
    k1 = k1_ref[...]

    @pl.when(b == 0)
    def _init():
        accv_s[...] = jnp.full((rows, chunk), neg_inf, jnp.float32)
        acci_s[...] = jnp.zeros((rows, chunk), jnp.int32)
        accl_s[...] = jnp.zeros((rows, chunk), jnp.float32)
        m_s[...] = jnp.full((rows, 1), neg_inf, jnp.float32)
        s_s[...] = jnp.zeros((rows, 1), jnp.float32)

    def body(c, carry):
        accv, acci, accl, m, s = carry
        start = c * chunk
        l = logits_ref[:, pl.ds(start, chunk)]
        gidx = lane0 + (base + start)
        valid = gidx < vocab

        o0, o1 = _tf_rounds(jnp.zeros((rows, chunk), jnp.uint32),
                            gidx.astype(jnp.uint32), k0, k1)
        bits = o0 ^ o1
        uf = pltpu.bitcast((bits >> jnp.uint32(9)) | jnp.uint32(0x3F800000),
                           jnp.float32) - jnp.float32(1.0)
        u = jnp.maximum(_TINY, uf + _TINY)
        g = -jnp.log(-jnp.log(u))

        lm = jnp.where(valid, l, neg_inf)
        v = jnp.where(valid, l + g, neg_inf)

        take = v > accv
        accv = jnp.maximum(v, accv)
        acci = jnp.where(take, gidx, acci)
        accl = jnp.where(take, l, accl)

        blm = jnp.max(lm, axis=1, keepdims=True)
        bs = jnp.sum(jnp.where(valid, jnp.exp(lm - blm), 0.0), axis=1,
                     keepdims=True)
        mn = jnp.maximum(m, blm)
        s = s * jnp.exp(m - mn) + bs * jnp.exp(blm - mn)
        return accv, acci, accl, mn, s

    carry0 = (accv_s[...], acci_s[...], accl_s[...], m_s[...], s_s[...])
    accv, acci, accl, m, s = jax.lax.fori_loop(0, nchunks, body, carry0,
                                               unroll=16)
    accv_s[...] = accv
    acci_s[...] = acci
    accl_s[...] = accl
    m_s[...] = m
    s_s[...] = s

    @pl.when(b == nb - 1)
    def _emit():
        bv = jnp.max(accv, axis=1, keepdims=True)
        tied = accv == bv
        big = jnp.int32(2**31 - 1)
        bi = jnp.min(jnp.where(tied, acci, big), axis=1, keepdims=True)
        win = tied & (acci == bi)
        bl = jnp.max(jnp.where(win, accl, neg_inf), axis=1, keepdims=True)
        act_ref[...] = bi
        lp_ref[...] = bl - (m + jnp.log(s))


def kernel(logits):
    n_fields, vocab = logits.shape
    rows = 8
    block = 8192
    chunk = 512
    nfb = math.ceil(n_fields / rows)
    nb = math.ceil(vocab / block)
    nfp = nfb * rows

    k0np, k1np = _field_keys(n_fields)
    k0 = jnp.asarray(np.pad(k0np, (0, nfp - n_fields)).reshape(nfp, 1))
    k1 = jnp.asarray(np.pad(k1np, (0, nfp - n_fields)).reshape(nfp, 1))

    act, lp = pl.pallas_call(
        functools.partial(_sample_kernel, block=block, chunk=chunk,
                          vocab=vocab, nb=nb),
        grid=(nfb, nb),
        in_specs=[
            pl.BlockSpec((rows, block), lambda f, b: (f, b)),
            pl.BlockSpec((rows, 1), lambda f, b: (f, 0)),
            pl.BlockSpec((rows, 1), lambda f, b: (f, 0)),
        ],
        out_specs=[
            pl.BlockSpec((rows, 1), lambda f, b: (f, 0)),
            pl.BlockSpec((rows, 1), lambda f, b: (f, 0)),
        ],
        out_shape=[
            jax.ShapeDtypeStruct((nfp, 1), jnp.int32),
            jax.ShapeDtypeStruct((nfp, 1), jnp.float32),
        ],
        scratch_shapes=[
            pltpu.VMEM((rows, chunk), jnp.float32),
            pltpu.VMEM((rows, chunk), jnp.int32),
            pltpu.VMEM((rows, chunk), jnp.float32),
            pltpu.VMEM((rows, 1), jnp.float32),
            pltpu.VMEM((rows, 1), jnp.float32),
        ],
        compiler_params=pltpu.CompilerParams(
            dimension_semantics=("parallel", "arbitrary")),
    )(logits, k0, k1)

    action = act[:n_fields, 0]
    log_prob = lp[:n_fields, 0].sum()
    return (action, log_prob, jnp.float32(1.0))


# speedup vs baseline: 2.3977x; 1.0436x over previous
import functools
import math

import jax
import jax.numpy as jnp
import numpy as np
from jax.experimental import pallas as pl
from jax.experimental.pallas import tpu as pltpu

_ROT_A = (13, 15, 26, 6)
_ROT_B = (17, 29, 16, 24)
_TINY = np.float32(np.finfo(np.float32).tiny)


def _np_threefry2x32(k0, k1, x0, x1):
    x0 = np.asarray(x0, np.uint32).copy()
    x1 = np.asarray(x1, np.uint32).copy()
    ks = [np.uint32(k0), np.uint32(k1),
          np.uint32(np.uint32(k0) ^ np.uint32(k1) ^ np.uint32(0x1BD11BDA))]
    rots = [_ROT_A, _ROT_B]
    x0 = (x0 + ks[0]).astype(np.uint32)
    x1 = (x1 + ks[1]).astype(np.uint32)
    for i in range(5):
        for r in rots[i % 2]:
            x0 = (x0 + x1).astype(np.uint32)
            x1 = ((x1 << np.uint32(r)) | (x1 >> np.uint32(32 - r))).astype(np.uint32)
            x1 = (x1 ^ x0).astype(np.uint32)
        x0 = (x0 + ks[(i + 1) % 3]).astype(np.uint32)
        x1 = (x1 + ks[(i + 2) % 3] + np.uint32(i + 1)).astype(np.uint32)
    return x0, x1


def _field_keys(n_fields):
    idx = np.arange(n_fields, dtype=np.uint32)
    o0, o1 = _np_threefry2x32(0, 42, np.zeros(n_fields, np.uint32), idx)
    return o0, o1


def _tf20(x0, x1, kx0, kx1):
    for i in range(5):
        rots = _ROT_A if i % 2 == 0 else _ROT_B
        for r in rots:
            x0 = x0 + x1
            x1 = (x1 << jnp.uint32(r)) | (x1 >> jnp.uint32(32 - r))
            x1 = x1 ^ x0
        x0 = x0 + kx0[i]
        x1 = x1 + kx1[i]
    return x0 ^ x1


def _sample_kernel(logits_ref, k0_ref, k1_ref, act_ref, lp_ref,
                   accv_s, acci_s, accl_s, m_s, sl_s,
                   *, block, chunk, vocab, nb):
    b = pl.program_id(1)
    rows = logits_ref.shape[0]
    nchunks = block // chunk
    base = b * block
    neg_inf = jnp.float32(-jnp.inf)
    lane0 = jax.lax.broadcasted_iota(jnp.int32, (rows, chunk), 1)

    k0 = k0_ref[...]
    k1 = k1_ref[...]
    k2 = k0 ^ k1 ^ jnp.uint32(0x1BD11BDA)
    ks = (k0, k1, k2)
    kx0 = tuple(ks[(i + 1) % 3] for i in range(5))
    kx1 = tuple(ks[(i + 2) % 3] + jnp.uint32(i + 1) for i in range(5))

    @pl.when(b == 0)
    def _init():
        accv_s[...] = jnp.full((rows, chunk), neg_inf, jnp.float32)
        acci_s[...] = jnp.zeros((rows, chunk), jnp.int32)
        accl_s[...] = jnp.zeros((rows, chunk), jnp.float32)
        m_s[...] = jnp.full((rows, 1), neg_inf, jnp.float32)
        sl_s[...] = jnp.zeros((rows, chunk), jnp.float32)

    def block_max(n, masked):
        def mbody(c, acc):
            l = logits_ref[:, pl.ds(c * chunk, chunk)]
            if masked:
                l = jnp.where(lane0 + (base + c * chunk) < vocab, l, neg_inf)
            return jnp.maximum(acc, l)
        bmax = jax.lax.fori_loop(
            0, n, mbody, jnp.full((rows, chunk), neg_inf, jnp.float32),
            unroll=4)
        return jnp.max(bmax, axis=1, keepdims=True)

    def run(n, masked):
        mp = m_s[...]
        mn = jnp.maximum(mp, block_max(n, masked))
        m_s[...] = mn

        def body(c, carry):
            accv, acci, accl, sl = carry
            start = c * chunk
            l = logits_ref[:, pl.ds(start, chunk)]
            gidx = lane0 + (base + start)
            bits = _tf20(k0, gidx.astype(jnp.uint32) + k1, kx0, kx1)
            uf = pltpu.bitcast(
                (bits >> jnp.uint32(9)) | jnp.uint32(0x3F800000),
                jnp.float32) - jnp.float32(1.0)
            g = -jnp.log(-jnp.log(uf + _TINY))
            v = l + g
            e = jnp.exp(l - mn)
            if masked:
                valid = gidx < vocab
                v = jnp.where(valid, v, neg_inf)
                e = jnp.where(valid, e, 0.0)
            take = v > accv
            accv = jnp.maximum(v, accv)
            acci = jnp.where(take, gidx, acci)
            accl = jnp.where(take, l, accl)
            sl = sl + e
            return accv, acci, accl, sl

        carry0 = (accv_s[...], acci_s[...], accl_s[...],
                  sl_s[...] * jnp.exp(mp - mn))
        accv, acci, accl, sl = jax.lax.fori_loop(0, n, body, carry0,
                                                 unroll=min(n, 16))
        accv_s[...] = accv
        acci_s[...] = acci
        accl_s[...] = accl
        sl_s[...] = sl
        return accv, acci, accl, sl, mn

    @pl.when(b < nb - 1)
    def _full():
        run(nchunks, masked=False)

    @pl.when(b == nb - 1)
    def _tail():
        ntail = math.ceil((vocab - (nb - 1) * block) / chunk)
        accv, acci, accl, sl, mn = run(ntail, masked=True)
        bv = jnp.max(accv, axis=1, keepdims=True)
        tied = accv == bv
        big = jnp.int32(2**31 - 1)
        bi = jnp.min(jnp.where(tied, acci, big), axis=1, keepdims=True)
        win = tied & (acci == bi)
        bl = jnp.max(jnp.where(win, accl, neg_inf), axis=1, keepdims=True)
        s = jnp.sum(sl, axis=1, keepdims=True)
        act_ref[...] = bi
        lp_ref[...] = bl - (mn + jnp.log(s))


def kernel(logits):
    n_fields, vocab = logits.shape
    rows = 8
    block = 8192
    chunk = 512
    nfb = math.ceil(n_fields / rows)
    nb = math.ceil(vocab / block)
    nfp = nfb * rows

    k0np, k1np = _field_keys(n_fields)
    k0 = jnp.asarray(np.pad(k0np, (0, nfp - n_fields)).reshape(nfp, 1))
    k1 = jnp.asarray(np.pad(k1np, (0, nfp - n_fields)).reshape(nfp, 1))

    act, lp = pl.pallas_call(
        functools.partial(_sample_kernel, block=block, chunk=chunk,
                          vocab=vocab, nb=nb),
        grid=(nfb, nb),
        in_specs=[
            pl.BlockSpec((rows, block), lambda f, b: (f, b)),
            pl.BlockSpec((rows, 1), lambda f, b: (f, 0)),
            pl.BlockSpec((rows, 1), lambda f, b: (f, 0)),
        ],
        out_specs=[
            pl.BlockSpec((rows, 1), lambda f, b: (f, 0)),
            pl.BlockSpec((rows, 1), lambda f, b: (f, 0)),
        ],
        out_shape=[
            jax.ShapeDtypeStruct((nfp, 1), jnp.int32),
            jax.ShapeDtypeStruct((nfp, 1), jnp.float32),
        ],
        scratch_shapes=[
            pltpu.VMEM((rows, chunk), jnp.float32),
            pltpu.VMEM((rows, chunk), jnp.int32),
            pltpu.VMEM((rows, chunk), jnp.float32),
            pltpu.VMEM((rows, 1), jnp.float32),
            pltpu.VMEM((rows, chunk), jnp.float32),
        ],
        compiler_params=pltpu.CompilerParams(
            dimension_semantics=("parallel", "arbitrary")),
    )(logits, k0, k1)

    action = act[:n_fields, 0]
    log_prob = lp[:n_fields, 0].sum()
    return (action, log_prob, jnp.float32(1.0))
